# baseline (speedup 1.0000x reference)
import jax
import jax.numpy as jnp
from jax.experimental import pallas as pl
from jax.experimental.pallas import tpu as pltpu

_CH = 256000
_NBUF = 16
_LAG = 8


def _ring_copy_body(u_src, i_src, u_dst, i_dst, buf, sem_in, sem_out):
    chunks = []
    for c in range(6400000 // _CH):
        chunks.append((u_src, u_dst, c * _CH))
    for c in range(64000000 // _CH):
        chunks.append((i_src, i_dst, c * _CH))
    T = len(chunks)

    def copy_in(c):
        s, _, off = chunks[c]
        b = c % _NBUF
        return pltpu.make_async_copy(s.at[pl.ds(off, _CH)], buf.at[b], sem_in.at[b])

    def copy_out(c):
        _, d, off = chunks[c]
        b = c % _NBUF
        return pltpu.make_async_copy(buf.at[b], d.at[pl.ds(off, _CH)], sem_out.at[b])

    out_waited = [False] * T
    for b in range(min(_NBUF, T)):
        copy_in(b).start()
    for c in range(T):
        r = c - _LAG
        if 0 <= r and r + _NBUF < T:
            copy_out(r).wait()
            out_waited[r] = True
            copy_in(r + _NBUF).start()
        copy_in(c).wait()
        copy_out(c).start()
    for c in range(T):
        if not out_waited[c]:
            copy_out(c).wait()


def kernel(W_user, W_item):
    u = W_user.reshape(-1)
    i = W_item.reshape(-1)
    out = pl.pallas_call(
        _ring_copy_body,
        in_specs=[
            pl.BlockSpec(memory_space=pltpu.HBM),
            pl.BlockSpec(memory_space=pltpu.HBM),
        ],
        out_specs=[
            pl.BlockSpec(memory_space=pltpu.HBM),
            pl.BlockSpec(memory_space=pltpu.HBM),
        ],
        out_shape=[
            jax.ShapeDtypeStruct(u.shape, u.dtype),
            jax.ShapeDtypeStruct(i.shape, i.dtype),
        ],
        scratch_shapes=[
            pltpu.VMEM((_NBUF, _CH), jnp.float32),
            pltpu.SemaphoreType.DMA((_NBUF,)),
            pltpu.SemaphoreType.DMA((_NBUF,)),
        ],
    )(u, i)
    return (out[0].reshape(W_user.shape), out[1].reshape(W_item.shape))


# speedup vs baseline: 1.3382x; 1.3382x over previous
import jax
import jax.numpy as jnp
from jax.experimental import pallas as pl
from jax.experimental.pallas import tpu as pltpu

_R = 10000
_NBUF = 12
_LAG = 6
_NPRI = 2


def _ring_copy_body(u_src, i_src, u_dst, i_dst, buf, sem_in, sem_out):
    chunks = []
    for c in range(100000 // _R):
        chunks.append((u_src, u_dst, c * _R))
    for c in range(1000000 // _R):
        chunks.append((i_src, i_dst, c * _R))
    T = len(chunks)

    def copy_in(c):
        s, _, off = chunks[c]
        b = c % _NBUF
        return pltpu.make_async_copy(s.at[pl.ds(off, _R), :], buf.at[b], sem_in.at[b])

    def copy_out(c):
        _, d, off = chunks[c]
        b = c % _NBUF
        return pltpu.make_async_copy(buf.at[b], d.at[pl.ds(off, _R), :], sem_out.at[b])

    out_waited = [False] * T
    for b in range(min(_NBUF, T)):
        copy_in(b).start(priority=b % _NPRI)
    for c in range(T):
        r = c - _LAG
        if 0 <= r and r + _NBUF < T:
            copy_out(r).wait()
            out_waited[r] = True
            copy_in(r + _NBUF).start(priority=(r + _NBUF) % _NPRI)
        copy_in(c).wait()
        copy_out(c).start(priority=c % _NPRI)
    for c in range(T):
        if not out_waited[c]:
            copy_out(c).wait()


def kernel(W_user, W_item):
    out = pl.pallas_call(
        _ring_copy_body,
        in_specs=[
            pl.BlockSpec(memory_space=pltpu.HBM),
            pl.BlockSpec(memory_space=pltpu.HBM),
        ],
        out_specs=[
            pl.BlockSpec(memory_space=pltpu.HBM),
            pl.BlockSpec(memory_space=pltpu.HBM),
        ],
        out_shape=[
            jax.ShapeDtypeStruct(W_user.shape, W_user.dtype),
            jax.ShapeDtypeStruct(W_item.shape, W_item.dtype),
        ],
        scratch_shapes=[
            pltpu.VMEM((_NBUF, _R, 64), jnp.float32),
            pltpu.SemaphoreType.DMA((_NBUF,)),
            pltpu.SemaphoreType.DMA((_NBUF,)),
        ],
    )(W_user, W_item)
    return (out[0], out[1])
